# baseline (speedup 1.0000x reference)
import functools
import math

import jax
import jax.numpy as jnp
from jax import lax
from jax.experimental import pallas as pl
from jax.experimental.pallas import tpu as pltpu
from jax.experimental.pallas import tpu_sc as plsc

_B = 4096
_L = 200
_D = 64
_LANES = 16
_NC = 2
_NS = 16
_NW = _NC * _NS
_BPW = _B // _NW
_BLK = _BPW * _D
_NB = 5
_PF = 4
_SCALE = math.sqrt(_D)


def _tec_body(x4_hbm, table_hbm, pos_hbm, out4_hbm, idx_v, pos_v, rows_v,
              tbuf_v, gsem, osem):
    wid = lax.axis_index("s") * _NC + lax.axis_index("c")

    pltpu.sync_copy(pos_hbm, pos_v)
    pltpu.sync_copy(x4_hbm.at[:, wid], idx_v)

    def gather_l(l, g):
        pltpu.async_copy(table_hbm.at[idx_v.at[l // 8, l % 8]],
                         rows_v.at[g], gsem.at[g])

    def wait_gather(g):
        pltpu.make_async_copy(table_hbm.at[idx_v.at[0, 0]],
                              rows_v.at[g], gsem.at[g]).wait()

    def store_l(l, s):
        for d8 in range(_D // 8):
            pltpu.async_copy(tbuf_v.at[s, pl.ds(d8 * 1024, 1024)],
                             out4_hbm.at[l, d8, wid], osem.at[s])

    def wait_store(s):
        for d8 in range(_D // 8):
            pltpu.make_async_copy(tbuf_v.at[s, pl.ds(d8 * 1024, 1024)],
                                  out4_hbm.at[0, 0, 0], osem.at[s]).wait()

    iota = lax.iota(jnp.int32, _LANES)
    ibs = [iota + mg * _LANES for mg in range(_BPW // _LANES)]

    for l in range(_PF):
        gather_l(l, l)

    @pl.loop(0, _L, step=_NB)
    def l_block(k):
        for s in range(_NB):
            l = k + s
            j = l + _PF

            @pl.when(j < _L)
            def _prefetch():
                gather_l(j, (s + _PF) % _NB)

            wait_gather(s)

            @pl.when(l >= _NB)
            def _drain():
                wait_store(s)

            rows = rows_v.at[s]
            tb = tbuf_v.at[s]
            pv = [pos_v[l, pl.ds(k4 * _LANES, _LANES)] for k4 in range(4)]

            def compute(dg, c):
                mk = (iota + dg) & (_LANES - 1)
                smi = mk * jnp.int32(_D * 2) + iota
                for k4 in range(4):
                    i_d = mk + k4 * _LANES
                    p = pv[k4][mk]
                    st0 = smi + jnp.int32(k4 * _LANES * _D * 2)
                    vs = [plsc.load_gather(rows, [ibs[mg], i_d])
                          for mg in range(_BPW // _LANES)]
                    for mg in range(_BPW // _LANES):
                        plsc.store_scatter(tb, [st0 + mg * _LANES],
                                           vs[mg] * _SCALE + p)
                return c

            lax.fori_loop(0, _LANES, compute, 0, unroll=2)
            store_l(l, s)

    for s in range(_NB):
        wait_store(s)


@jax.jit
def _run(x4, table, pos_encoding):
    mesh = plsc.VectorSubcoreMesh(core_axis_name="c", subcore_axis_name="s")
    f = pl.kernel(
        _tec_body,
        out_type=jax.ShapeDtypeStruct((_L, _D // 8, _NW, 8 * 128),
                                      jnp.float32),
        mesh=mesh,
        scratch_types=[
            pltpu.VMEM((_L // 8, 8, _BPW), jnp.int32),
            pltpu.VMEM((_L, _D), jnp.float32),
            pltpu.VMEM((_NB, _BPW, _D), jnp.float32),
            pltpu.VMEM((_NB, _BLK), jnp.float32),
            pltpu.SemaphoreType.DMA((_NB,)),
            pltpu.SemaphoreType.DMA((_NB,)),
        ],
        compiler_params=pltpu.CompilerParams(use_tc_tiling_on_sc=False,
                                             needs_layout_passes=False),
    )
    return f(x4, table, pos_encoding)


def kernel(x, table, pos_encoding):
    x4 = (x.astype(jnp.int32).transpose(1, 0)
          .reshape(_L // 8, 8, _NW, _BPW).transpose(0, 2, 1, 3))
    out4 = _run(x4, table, pos_encoding)
    out5 = out4.reshape(_L, _D // 8, _NW, 8, 128)
    return out5.transpose(2, 4, 0, 1, 3).reshape(_B, _L, _D)


# speedup vs baseline: 1.3871x; 1.3871x over previous
import functools
import math

import jax
import jax.numpy as jnp
from jax import lax
from jax.experimental import pallas as pl
from jax.experimental.pallas import tpu as pltpu
from jax.experimental.pallas import tpu_sc as plsc

_B = 4096
_L = 200
_D = 64
_LANES = 16
_NC = 2
_NS = 16
_NW = _NC * _NS
_BPW = _B // _NW
_BLK = _BPW * _D
_NB = 4
_PF = 3
_SCALE = math.sqrt(_D)


def _tec_body(x4_hbm, table_hbm, pos_hbm, out4_hbm, idx_v, pos_v, rows_v,
              tbuf_v, gsem, osem):
    wid = lax.axis_index("s") * _NC + lax.axis_index("c")

    pltpu.sync_copy(pos_hbm, pos_v)
    pltpu.sync_copy(x4_hbm.at[:, wid], idx_v)

    def gather_l(l, g):
        pltpu.async_copy(table_hbm.at[idx_v.at[l // 8, l % 8]],
                         rows_v.at[g], gsem.at[g])

    def wait_gather(g):
        pltpu.make_async_copy(table_hbm.at[idx_v.at[0, 0]],
                              rows_v.at[g], gsem.at[g]).wait()

    def store_l(l, s):
        for d8 in range(_D // 8):
            pltpu.async_copy(tbuf_v.at[s, pl.ds(d8 * 1024, 1024)],
                             out4_hbm.at[l, d8, wid], osem.at[s])

    def wait_store(s):
        for d8 in range(_D // 8):
            pltpu.make_async_copy(tbuf_v.at[s, pl.ds(d8 * 1024, 1024)],
                                  out4_hbm.at[0, 0, 0], osem.at[s]).wait()

    iota = lax.iota(jnp.int32, _LANES)
    ibs = [iota + mg * _LANES for mg in range(_BPW // _LANES)]

    for l in range(_PF):
        gather_l(l, l)

    @pl.loop(0, _L, step=_NB)
    def l_block(k):
        for s in range(_NB):
            l = k + s
            j = l + _PF

            @pl.when(j < _L)
            def _prefetch():
                gather_l(j, (s + _PF) % _NB)

            wait_gather(s)

            @pl.when(l >= _NB)
            def _drain():
                wait_store(s)

            rows = rows_v.at[s]
            tb = tbuf_v.at[s]
            pv = [pos_v[l, pl.ds(k4 * _LANES, _LANES)] for k4 in range(4)]

            def compute(dg, c):
                mk = (iota + dg) & (_LANES - 1)
                smi = mk * jnp.int32(_D * 2) + iota
                for k2 in range(2):
                    vs = []
                    for k4 in (2 * k2, 2 * k2 + 1):
                        i_d = mk + k4 * _LANES
                        vs += [plsc.load_gather(rows, [ibs[mg], i_d])
                               for mg in range(_BPW // _LANES)]
                    for h, k4 in enumerate((2 * k2, 2 * k2 + 1)):
                        p = pv[k4][mk]
                        st0 = smi + jnp.int32(k4 * _LANES * _D * 2)
                        for mg in range(_BPW // _LANES):
                            plsc.store_scatter(
                                tb, [st0 + mg * _LANES],
                                vs[h * 8 + mg] * _SCALE + p)
                return c

            lax.fori_loop(0, _LANES, compute, 0, unroll=2)
            store_l(l, s)

    for s in range(_NB):
        wait_store(s)


@jax.jit
def _run(x4, table, pos_encoding):
    mesh = plsc.VectorSubcoreMesh(core_axis_name="c", subcore_axis_name="s")
    f = pl.kernel(
        _tec_body,
        out_type=jax.ShapeDtypeStruct((_L, _D // 8, _NW, 8 * 128),
                                      jnp.float32),
        mesh=mesh,
        scratch_types=[
            pltpu.VMEM((_L // 8, 8, _BPW), jnp.int32),
            pltpu.VMEM((_L, _D), jnp.float32),
            pltpu.VMEM((_NB, _BPW, _D), jnp.float32),
            pltpu.VMEM((_NB, _BLK), jnp.float32),
            pltpu.SemaphoreType.DMA((_NB,)),
            pltpu.SemaphoreType.DMA((_NB,)),
        ],
        compiler_params=pltpu.CompilerParams(use_tc_tiling_on_sc=False,
                                             needs_layout_passes=False),
    )
    return f(x4, table, pos_encoding)


def kernel(x, table, pos_encoding):
    x4 = (x.astype(jnp.int32).transpose(1, 0)
          .reshape(_L // 8, 8, _NW, _BPW).transpose(0, 2, 1, 3))
    out4 = _run(x4, table, pos_encoding)
    out5 = out4.reshape(_L, _D // 8, _NW, 8, 128)
    return out5.transpose(2, 4, 0, 1, 3).reshape(_B, _L, _D)
